# baseline (speedup 1.0000x reference)
import functools

import jax
import jax.numpy as jnp
from jax import lax
from jax.experimental import pallas as pl
from jax.experimental.pallas import tpu as pltpu
from jax.experimental.pallas import tpu_sc as plsc

NC = 2
NS = 16
LANES = 16


def _tc_layer1(x, W1, a_src, a_dst, blk):
    n, in_dim = x.shape
    hid = W1.shape[1]
    grid = n // blk

    def body(x_ref, w_ref, asrc_ref, adst_ref, h_ref, es_ref, ed_ref):
        h = jnp.dot(x_ref[...], w_ref[...], preferred_element_type=jnp.float32)
        h_ref[...] = h
        es_ref[...] = jnp.sum(h * asrc_ref[...], axis=1, keepdims=True)
        ed_ref[...] = jnp.sum(h * adst_ref[...], axis=1, keepdims=True)

    return pl.pallas_call(
        body,
        grid=(grid,),
        in_specs=[
            pl.BlockSpec((blk, in_dim), lambda i: (i, 0)),
            pl.BlockSpec((in_dim, hid), lambda i: (0, 0)),
            pl.BlockSpec((1, hid), lambda i: (0, 0)),
            pl.BlockSpec((1, hid), lambda i: (0, 0)),
        ],
        out_specs=[
            pl.BlockSpec((blk, hid), lambda i: (i, 0)),
            pl.BlockSpec((blk, 1), lambda i: (i, 0)),
            pl.BlockSpec((blk, 1), lambda i: (i, 0)),
        ],
        out_shape=[
            jax.ShapeDtypeStruct((n, hid), jnp.float32),
            jax.ShapeDtypeStruct((n, 1), jnp.float32),
            jax.ShapeDtypeStruct((n, 1), jnp.float32),
        ],
    )(x, W1, a_src, a_dst)


def _tc_layer2(acc, s0, s1, b1, W2, a_src, a_dst, blk):
    n = acc.shape[1]
    hid = acc.shape[2]
    out = W2.shape[1]
    grid = n // blk

    def body(acc_ref, s0_ref, s1_ref, b_ref, w_ref, asrc_ref, adst_ref,
             h_ref, es_ref, ed_ref):
        num = acc_ref[0] + acc_ref[1]
        den = s0_ref[...] + s1_ref[...] + 1e-16
        z = num / den + b_ref[...]
        z = jnp.where(z > 0, z, jnp.exp(z) - 1.0)
        h = jnp.dot(z, w_ref[...], preferred_element_type=jnp.float32)
        h_ref[...] = h
        es_ref[...] = jnp.sum(h * asrc_ref[...], axis=1, keepdims=True)
        ed_ref[...] = jnp.sum(h * adst_ref[...], axis=1, keepdims=True)

    return pl.pallas_call(
        body,
        grid=(grid,),
        in_specs=[
            pl.BlockSpec((NC, blk, hid), lambda i: (0, i, 0)),
            pl.BlockSpec((blk, 1), lambda i: (i, 0)),
            pl.BlockSpec((blk, 1), lambda i: (i, 0)),
            pl.BlockSpec((1, hid), lambda i: (0, 0)),
            pl.BlockSpec((hid, out), lambda i: (0, 0)),
            pl.BlockSpec((1, out), lambda i: (0, 0)),
            pl.BlockSpec((1, out), lambda i: (0, 0)),
        ],
        out_specs=[
            pl.BlockSpec((blk, out), lambda i: (i, 0)),
            pl.BlockSpec((blk, 1), lambda i: (i, 0)),
            pl.BlockSpec((blk, 1), lambda i: (i, 0)),
        ],
        out_shape=[
            jax.ShapeDtypeStruct((n, out), jnp.float32),
            jax.ShapeDtypeStruct((n, 1), jnp.float32),
            jax.ShapeDtypeStruct((n, 1), jnp.float32),
        ],
    )(acc, s0, s1, b1, W2, a_src, a_dst)


def _tc_final(acc, s0, s1, b2, blk):
    n = acc.shape[1]
    out = acc.shape[2]
    grid = n // blk

    def body(acc_ref, s0_ref, s1_ref, b_ref, o_ref):
        o = (acc_ref[0] + acc_ref[1]) / (s0_ref[...] + s1_ref[...] + 1e-16)
        o = o + b_ref[...]
        m = jnp.max(o, axis=1, keepdims=True)
        lse = m + jnp.log(jnp.sum(jnp.exp(o - m), axis=1, keepdims=True))
        o_ref[...] = o - lse

    return pl.pallas_call(
        body,
        grid=(grid,),
        in_specs=[
            pl.BlockSpec((NC, blk, out), lambda i: (0, i, 0)),
            pl.BlockSpec((blk, 1), lambda i: (i, 0)),
            pl.BlockSpec((blk, 1), lambda i: (i, 0)),
            pl.BlockSpec((1, out), lambda i: (0, 0)),
        ],
        out_specs=pl.BlockSpec((blk, out), lambda i: (i, 0)),
        out_shape=jax.ShapeDtypeStruct((n, out), jnp.float32),
    )(acc, s0, s1, b2)


def _sc_msgpass(h, es, ed, edge_index):
    n, c = h.shape
    e = edge_index.shape[1]
    nw = NC * NS
    ept = e // nw
    ch = 80
    nch = ept // ch
    npair = (nch + 1) // 2
    rpt = 624
    tail = n - NS * rpt
    cr = c // LANES
    zrows = 64

    mesh = plsc.VectorSubcoreMesh(core_axis_name="c", subcore_axis_name="s",
                                  num_cores=NC, num_subcores=NS)

    @functools.partial(
        pl.kernel,
        out_type=[
            jax.ShapeDtypeStruct((NC, n, c), jnp.float32),
            jax.ShapeDtypeStruct((n,), jnp.float32),
            jax.ShapeDtypeStruct((n,), jnp.float32),
        ],
        mesh=mesh,
        compiler_params=pltpu.CompilerParams(needs_layout_passes=False,
                                             use_tc_tiling_on_sc=False),
        scratch_types=[
            pltpu.VMEM((n,), jnp.float32),
            pltpu.VMEM((n,), jnp.float32),
            [pltpu.VMEM((ch,), jnp.float32)] * 2,
            [pltpu.VMEM((2, ch), jnp.int32)] * 2,
            [pltpu.VMEM((ch, c), jnp.float32)] * 2,
            pltpu.VMEM((rpt + tail,), jnp.float32),
            pltpu.VMEM_SHARED((n, c), jnp.float32),
            pltpu.VMEM_SHARED((n,), jnp.float32),
            [pltpu.SemaphoreType.DMA] * 2,
            [pltpu.SemaphoreType.DMA] * 2,
            [pltpu.SemaphoreType.DMA] * 2,
            pltpu.SemaphoreType.DMA,
        ],
    )
    def k(h_hbm, es_hbm, ed_hbm, ei_hbm, acc_hbm, s0_hbm, s1_hbm,
          es_v, ed_v, exbuf, ibuf, hbuf,
          zs, acc_sh, s_sh, gsem, ssem, esem, isem):
        cid = lax.axis_index("c")
        sid = lax.axis_index("s")
        wid = cid * NS + sid
        ebase = wid * ept
        zero16 = jnp.zeros((LANES,), jnp.float32)

        def idx_src(g):
            return ei_hbm.at[:, pl.ds(ebase + g * ch, ch)]

        def issue_gather(b):
            pltpu.async_copy(h_hbm.at[ibuf[b].at[0]], hbuf[b], gsem[b])

        pltpu.async_copy(es_hbm, es_v, esem[0])
        pltpu.async_copy(ed_hbm, ed_v, esem[1])
        pltpu.async_copy(idx_src(0), ibuf[0], isem)

        def zzs(i, carry):
            zs[pl.ds(i * LANES, LANES)] = zero16
            return carry
        lax.fori_loop(0, (rpt + tail) // LANES, zzs, 0)

        def zzr(i, carry):
            for r in range(cr):
                hbuf[1][i, pl.ds(r * LANES, LANES)] = zero16
            return carry
        lax.fori_loop(0, zrows, zzr, 0)

        nz = rpt // zrows
        rem = rpt % zrows
        for i in range(nz):
            pltpu.async_copy(hbuf[1].at[pl.ds(0, zrows)],
                             acc_sh.at[pl.ds(sid * rpt + i * zrows, zrows)],
                             ssem[0])
        if rem:
            pltpu.async_copy(hbuf[1].at[pl.ds(0, rem)],
                             acc_sh.at[pl.ds(sid * rpt + nz * zrows, rem)],
                             ssem[0])
        pltpu.async_copy(zs.at[pl.ds(0, rpt)],
                         s_sh.at[pl.ds(sid * rpt, rpt)], ssem[1])

        @pl.when(sid == NS - 1)
        def _():
            pltpu.async_copy(hbuf[1].at[pl.ds(0, tail)],
                             acc_sh.at[pl.ds(NS * rpt, tail)], ssem[0])
            pltpu.async_copy(zs.at[pl.ds(0, tail)],
                             s_sh.at[pl.ds(NS * rpt, tail)], ssem[1])

        for i in range(nz):
            pltpu.make_async_copy(hbuf[1].at[pl.ds(0, zrows)],
                                  acc_sh.at[pl.ds(sid * rpt + i * zrows,
                                                  zrows)], ssem[0]).wait()
        if rem:
            pltpu.make_async_copy(hbuf[1].at[pl.ds(0, rem)],
                                  acc_sh.at[pl.ds(sid * rpt + nz * zrows,
                                                  rem)], ssem[0]).wait()
        pltpu.make_async_copy(zs.at[pl.ds(0, rpt)],
                              s_sh.at[pl.ds(sid * rpt, rpt)], ssem[1]).wait()

        @pl.when(sid == NS - 1)
        def _():
            pltpu.make_async_copy(hbuf[1].at[pl.ds(0, tail)],
                                  acc_sh.at[pl.ds(NS * rpt, tail)],
                                  ssem[0]).wait()
            pltpu.make_async_copy(zs.at[pl.ds(0, tail)],
                                  s_sh.at[pl.ds(NS * rpt, tail)],
                                  ssem[1]).wait()

        pltpu.make_async_copy(idx_src(0), ibuf[0], isem).wait()
        issue_gather(0)
        pltpu.make_async_copy(es_hbm, es_v, esem[0]).wait()
        pltpu.make_async_copy(ed_hbm, ed_v, esem[1]).wait()

        plsc.subcore_barrier()

        def compute_ex(b):
            @plsc.parallel_loop(0, ch // LANES, unroll=2)
            def lanes(j):
                sidx = ibuf[b][0, pl.ds(j * LANES, LANES)]
                didx = ibuf[b][1, pl.ds(j * LANES, LANES)]
                ea = plsc.load_gather(es_v, [sidx])
                eb = plsc.load_gather(ed_v, [didx])
                ee = ea + eb
                ee = jnp.where(ee >= 0, ee, 0.2 * ee)
                exbuf[b][pl.ds(j * LANES, LANES)] = jnp.exp(ee)

        def scale(b):
            @plsc.parallel_loop(0, ch // LANES, unroll=5)
            def sbody(j):
                exvec = exbuf[b][pl.ds(j * LANES, LANES)]
                for i in range(LANES):
                    al = exvec[i]
                    row = j * LANES + i
                    for r in range(cr):
                        sl = pl.ds(r * LANES, LANES)
                        hbuf[b][row, sl] = hbuf[b][row, sl] * al

        def wait_gather(b):
            pltpu.make_async_copy(h_hbm.at[ibuf[b].at[0]], hbuf[b],
                                  gsem[b]).wait()

        def wait_scatters(b):
            pltpu.make_async_copy(hbuf[b], acc_sh.at[ibuf[b].at[1]],
                                  ssem[b]).wait()
            pltpu.make_async_copy(exbuf[b], s_sh.at[ibuf[b].at[1]],
                                  esem[b]).wait()

        def pair(g2, carry):
            for b in (0, 1):
                g = 2 * g2 + b
                nb = 1 - b

                @pl.when(g < nch)
                def _():
                    @pl.when((g >= 1) & (g + 1 < nch))
                    def _():
                        wait_scatters(nb)

                    @pl.when(g + 1 < nch)
                    def _():
                        pltpu.async_copy(idx_src(g + 1), ibuf[nb], isem)
                    compute_ex(b)
                    wait_gather(b)
                    scale(b)

                    @pl.when(g + 1 < nch)
                    def _():
                        pltpu.make_async_copy(idx_src(g + 1), ibuf[nb],
                                              isem).wait()
                        issue_gather(nb)
                    pltpu.async_copy(hbuf[b], acc_sh.at[ibuf[b].at[1]],
                                     ssem[b], add=True)
                    pltpu.async_copy(exbuf[b], s_sh.at[ibuf[b].at[1]],
                                     esem[b], add=True)
            return carry
        lax.fori_loop(0, npair, pair, 0)

        wait_scatters((nch - 1) % 2)
        wait_scatters(nch % 2)

        plsc.subcore_barrier()

        chunks = [(i * zrows, zrows) for i in range(rpt // zrows)]
        if rpt % zrows:
            chunks.append(((rpt // zrows) * zrows, rpt % zrows))
        for i, (off, ln) in enumerate(chunks):
            b = i % 2
            if i >= 2:
                poff, pln = chunks[i - 2]
                pltpu.make_async_copy(
                    hbuf[b].at[pl.ds(0, pln)],
                    acc_hbm.at[cid, pl.ds(sid * rpt + poff, pln)],
                    gsem[b]).wait()
            pltpu.sync_copy(acc_sh.at[pl.ds(sid * rpt + off, ln)],
                            hbuf[b].at[pl.ds(0, ln)])
            pltpu.async_copy(hbuf[b].at[pl.ds(0, ln)],
                             acc_hbm.at[cid, pl.ds(sid * rpt + off, ln)],
                             gsem[b])
        for i in (len(chunks) - 2, len(chunks) - 1):
            off, ln = chunks[i]
            pltpu.make_async_copy(
                hbuf[i % 2].at[pl.ds(0, ln)],
                acc_hbm.at[cid, pl.ds(sid * rpt + off, ln)],
                gsem[i % 2]).wait()
        pltpu.sync_copy(s_sh.at[pl.ds(sid * rpt, rpt)], zs.at[pl.ds(0, rpt)])
        for core, s_hbm in ((0, s0_hbm), (1, s1_hbm)):
            @pl.when(cid == core)
            def _():
                pltpu.sync_copy(zs.at[pl.ds(0, rpt)],
                                s_hbm.at[pl.ds(sid * rpt, rpt)])

        @pl.when(sid == NS - 1)
        def _():
            pltpu.sync_copy(acc_sh.at[pl.ds(NS * rpt, tail)],
                            hbuf[0].at[pl.ds(0, tail)])
            pltpu.sync_copy(hbuf[0].at[pl.ds(0, tail)],
                            acc_hbm.at[cid, pl.ds(NS * rpt, tail)])
            pltpu.sync_copy(s_sh.at[pl.ds(NS * rpt, tail)],
                            zs.at[pl.ds(0, tail)])
            for core, s_hbm in ((0, s0_hbm), (1, s1_hbm)):
                @pl.when(cid == core)
                def _():
                    pltpu.sync_copy(zs.at[pl.ds(0, tail)],
                                    s_hbm.at[pl.ds(NS * rpt, tail)])

    return k(h, es, ed, edge_index)


def kernel(x, edge_index, W1, att_src1, att_dst1, b1, W2, att_src2, att_dst2, b2):
    n = x.shape[0]
    hid = W1.shape[1]
    out = W2.shape[1]
    blk = 1000

    h1, es1, ed1 = _tc_layer1(x, W1, att_src1, att_dst1, blk)
    acc1, s1a, s1b = _sc_msgpass(h1, es1.reshape(n), ed1.reshape(n), edge_index)
    h2, es2, ed2 = _tc_layer2(acc1, s1a.reshape(n, 1), s1b.reshape(n, 1),
                              b1.reshape(1, hid), W2, att_src2, att_dst2, blk)
    acc2, s2a, s2b = _sc_msgpass(h2, es2.reshape(n), ed2.reshape(n), edge_index)
    return _tc_final(acc2, s2a.reshape(n, 1), s2b.reshape(n, 1),
                     b2.reshape(1, out), blk)


# speedup vs baseline: 1.0016x; 1.0016x over previous
import functools

import jax
import jax.numpy as jnp
from jax import lax
from jax.experimental import pallas as pl
from jax.experimental.pallas import tpu as pltpu
from jax.experimental.pallas import tpu_sc as plsc

NC = 2
NS = 16
LANES = 16


def _tc_layer1(x, W1, a_src, a_dst, blk):
    n, in_dim = x.shape
    hid = W1.shape[1]
    grid = n // blk

    def body(x_ref, w_ref, asrc_ref, adst_ref, h_ref, es_ref, ed_ref):
        h = jnp.dot(x_ref[...], w_ref[...], preferred_element_type=jnp.float32)
        h_ref[...] = h
        es_ref[...] = jnp.sum(h * asrc_ref[...], axis=1, keepdims=True)
        ed_ref[...] = jnp.sum(h * adst_ref[...], axis=1, keepdims=True)

    return pl.pallas_call(
        body,
        grid=(grid,),
        in_specs=[
            pl.BlockSpec((blk, in_dim), lambda i: (i, 0)),
            pl.BlockSpec((in_dim, hid), lambda i: (0, 0)),
            pl.BlockSpec((1, hid), lambda i: (0, 0)),
            pl.BlockSpec((1, hid), lambda i: (0, 0)),
        ],
        out_specs=[
            pl.BlockSpec((blk, hid), lambda i: (i, 0)),
            pl.BlockSpec((blk, 1), lambda i: (i, 0)),
            pl.BlockSpec((blk, 1), lambda i: (i, 0)),
        ],
        out_shape=[
            jax.ShapeDtypeStruct((n, hid), jnp.float32),
            jax.ShapeDtypeStruct((n, 1), jnp.float32),
            jax.ShapeDtypeStruct((n, 1), jnp.float32),
        ],
    )(x, W1, a_src, a_dst)


def _tc_layer2(acc, s0, s1, b1, W2, a_src, a_dst, blk):
    n = acc.shape[1]
    hid = acc.shape[2]
    out = W2.shape[1]
    grid = n // blk

    def body(acc_ref, s0_ref, s1_ref, b_ref, w_ref, asrc_ref, adst_ref,
             h_ref, es_ref, ed_ref):
        num = acc_ref[0] + acc_ref[1]
        den = s0_ref[...] + s1_ref[...] + 1e-16
        z = num / den + b_ref[...]
        z = jnp.where(z > 0, z, jnp.exp(z) - 1.0)
        h = jnp.dot(z, w_ref[...], preferred_element_type=jnp.float32)
        h_ref[...] = h
        es_ref[...] = jnp.sum(h * asrc_ref[...], axis=1, keepdims=True)
        ed_ref[...] = jnp.sum(h * adst_ref[...], axis=1, keepdims=True)

    return pl.pallas_call(
        body,
        grid=(grid,),
        in_specs=[
            pl.BlockSpec((NC, blk, hid), lambda i: (0, i, 0)),
            pl.BlockSpec((blk, 1), lambda i: (i, 0)),
            pl.BlockSpec((blk, 1), lambda i: (i, 0)),
            pl.BlockSpec((1, hid), lambda i: (0, 0)),
            pl.BlockSpec((hid, out), lambda i: (0, 0)),
            pl.BlockSpec((1, out), lambda i: (0, 0)),
            pl.BlockSpec((1, out), lambda i: (0, 0)),
        ],
        out_specs=[
            pl.BlockSpec((blk, out), lambda i: (i, 0)),
            pl.BlockSpec((blk, 1), lambda i: (i, 0)),
            pl.BlockSpec((blk, 1), lambda i: (i, 0)),
        ],
        out_shape=[
            jax.ShapeDtypeStruct((n, out), jnp.float32),
            jax.ShapeDtypeStruct((n, 1), jnp.float32),
            jax.ShapeDtypeStruct((n, 1), jnp.float32),
        ],
    )(acc, s0, s1, b1, W2, a_src, a_dst)


def _tc_final(acc, s0, s1, b2, blk):
    n = acc.shape[1]
    out = acc.shape[2]
    grid = n // blk

    def body(acc_ref, s0_ref, s1_ref, b_ref, o_ref):
        o = (acc_ref[0] + acc_ref[1]) / (s0_ref[...] + s1_ref[...] + 1e-16)
        o = o + b_ref[...]
        m = jnp.max(o, axis=1, keepdims=True)
        lse = m + jnp.log(jnp.sum(jnp.exp(o - m), axis=1, keepdims=True))
        o_ref[...] = o - lse

    return pl.pallas_call(
        body,
        grid=(grid,),
        in_specs=[
            pl.BlockSpec((NC, blk, out), lambda i: (0, i, 0)),
            pl.BlockSpec((blk, 1), lambda i: (i, 0)),
            pl.BlockSpec((blk, 1), lambda i: (i, 0)),
            pl.BlockSpec((1, out), lambda i: (0, 0)),
        ],
        out_specs=pl.BlockSpec((blk, out), lambda i: (i, 0)),
        out_shape=jax.ShapeDtypeStruct((n, out), jnp.float32),
    )(acc, s0, s1, b2)


def _sc_msgpass(h, es, ed, edge_index):
    n, c = h.shape
    e = edge_index.shape[1]
    nw = NC * NS
    ept = e // nw
    ch = 80
    nch = ept // ch
    npair = (nch + 1) // 2
    rpt = 624
    tail = n - NS * rpt
    cr = c // LANES
    zrows = 64

    mesh = plsc.VectorSubcoreMesh(core_axis_name="c", subcore_axis_name="s",
                                  num_cores=NC, num_subcores=NS)

    @functools.partial(
        pl.kernel,
        out_type=[
            jax.ShapeDtypeStruct((NC, n, c), jnp.float32),
            jax.ShapeDtypeStruct((n,), jnp.float32),
            jax.ShapeDtypeStruct((n,), jnp.float32),
        ],
        mesh=mesh,
        compiler_params=pltpu.CompilerParams(needs_layout_passes=False,
                                             use_tc_tiling_on_sc=False),
        scratch_types=[
            pltpu.VMEM((n,), jnp.float32),
            pltpu.VMEM((n,), jnp.float32),
            [pltpu.VMEM((ch,), jnp.float32)] * 2,
            [pltpu.VMEM((2, ch), jnp.int32)] * 2,
            [pltpu.VMEM((ch, c), jnp.float32)] * 2,
            pltpu.VMEM((rpt + tail,), jnp.float32),
            pltpu.VMEM_SHARED((n, c), jnp.float32),
            pltpu.VMEM_SHARED((n,), jnp.float32),
            [pltpu.SemaphoreType.DMA] * 2,
            [pltpu.SemaphoreType.DMA] * 2,
            [pltpu.SemaphoreType.DMA] * 2,
            pltpu.SemaphoreType.DMA,
        ],
    )
    def k(h_hbm, es_hbm, ed_hbm, ei_hbm, acc_hbm, s0_hbm, s1_hbm,
          es_v, ed_v, exbuf, ibuf, hbuf,
          zs, acc_sh, s_sh, gsem, ssem, esem, isem):
        cid = lax.axis_index("c")
        sid = lax.axis_index("s")
        wid = cid * NS + sid
        ebase = wid * ept
        zero16 = jnp.zeros((LANES,), jnp.float32)

        def idx_src(g):
            return ei_hbm.at[:, pl.ds(ebase + g * ch, ch)]

        def issue_gather(b):
            pltpu.async_copy(h_hbm.at[ibuf[b].at[0]], hbuf[b], gsem[b])

        pltpu.async_copy(es_hbm, es_v, esem[0])
        pltpu.async_copy(ed_hbm, ed_v, esem[1])
        pltpu.async_copy(idx_src(0), ibuf[0], isem)

        def zzs(i, carry):
            zs[pl.ds(i * LANES, LANES)] = zero16
            return carry
        lax.fori_loop(0, (rpt + tail) // LANES, zzs, 0)

        def zzr(i, carry):
            for r in range(cr):
                hbuf[1][i, pl.ds(r * LANES, LANES)] = zero16
            return carry
        lax.fori_loop(0, zrows, zzr, 0)

        nz = rpt // zrows
        rem = rpt % zrows
        for i in range(nz):
            pltpu.async_copy(hbuf[1].at[pl.ds(0, zrows)],
                             acc_sh.at[pl.ds(sid * rpt + i * zrows, zrows)],
                             ssem[0])
        if rem:
            pltpu.async_copy(hbuf[1].at[pl.ds(0, rem)],
                             acc_sh.at[pl.ds(sid * rpt + nz * zrows, rem)],
                             ssem[0])
        pltpu.async_copy(zs.at[pl.ds(0, rpt)],
                         s_sh.at[pl.ds(sid * rpt, rpt)], ssem[1])

        @pl.when(sid == NS - 1)
        def _():
            pltpu.async_copy(hbuf[1].at[pl.ds(0, tail)],
                             acc_sh.at[pl.ds(NS * rpt, tail)], ssem[0])
            pltpu.async_copy(zs.at[pl.ds(0, tail)],
                             s_sh.at[pl.ds(NS * rpt, tail)], ssem[1])

        for i in range(nz):
            pltpu.make_async_copy(hbuf[1].at[pl.ds(0, zrows)],
                                  acc_sh.at[pl.ds(sid * rpt + i * zrows,
                                                  zrows)], ssem[0]).wait()
        if rem:
            pltpu.make_async_copy(hbuf[1].at[pl.ds(0, rem)],
                                  acc_sh.at[pl.ds(sid * rpt + nz * zrows,
                                                  rem)], ssem[0]).wait()
        pltpu.make_async_copy(zs.at[pl.ds(0, rpt)],
                              s_sh.at[pl.ds(sid * rpt, rpt)], ssem[1]).wait()

        @pl.when(sid == NS - 1)
        def _():
            pltpu.make_async_copy(hbuf[1].at[pl.ds(0, tail)],
                                  acc_sh.at[pl.ds(NS * rpt, tail)],
                                  ssem[0]).wait()
            pltpu.make_async_copy(zs.at[pl.ds(0, tail)],
                                  s_sh.at[pl.ds(NS * rpt, tail)],
                                  ssem[1]).wait()

        pltpu.make_async_copy(idx_src(0), ibuf[0], isem).wait()
        issue_gather(0)
        pltpu.make_async_copy(es_hbm, es_v, esem[0]).wait()
        pltpu.make_async_copy(ed_hbm, ed_v, esem[1]).wait()

        plsc.subcore_barrier()

        def compute_ex(b):
            @plsc.parallel_loop(0, ch // LANES, unroll=2)
            def lanes(j):
                sidx = ibuf[b][0, pl.ds(j * LANES, LANES)]
                didx = ibuf[b][1, pl.ds(j * LANES, LANES)]
                ea = plsc.load_gather(es_v, [sidx])
                eb = plsc.load_gather(ed_v, [didx])
                ee = ea + eb
                ee = jnp.where(ee >= 0, ee, 0.2 * ee)
                exbuf[b][pl.ds(j * LANES, LANES)] = jnp.exp(ee)

        def scale(b):
            @plsc.parallel_loop(0, ch // LANES, unroll=2)
            def sbody(j):
                exvec = exbuf[b][pl.ds(j * LANES, LANES)]
                for i in range(LANES):
                    al = exvec[i]
                    row = j * LANES + i
                    for r in range(cr):
                        sl = pl.ds(r * LANES, LANES)
                        hbuf[b][row, sl] = hbuf[b][row, sl] * al

        def wait_gather(b):
            pltpu.make_async_copy(h_hbm.at[ibuf[b].at[0]], hbuf[b],
                                  gsem[b]).wait()

        def wait_scatters(b):
            pltpu.make_async_copy(hbuf[b], acc_sh.at[ibuf[b].at[1]],
                                  ssem[b]).wait()
            pltpu.make_async_copy(exbuf[b], s_sh.at[ibuf[b].at[1]],
                                  esem[b]).wait()

        def pair(g2, carry):
            for b in (0, 1):
                g = 2 * g2 + b
                nb = 1 - b

                @pl.when(g < nch)
                def _():
                    @pl.when((g >= 1) & (g + 1 < nch))
                    def _():
                        wait_scatters(nb)

                    @pl.when(g + 1 < nch)
                    def _():
                        pltpu.async_copy(idx_src(g + 1), ibuf[nb], isem)
                    compute_ex(b)
                    wait_gather(b)
                    scale(b)

                    @pl.when(g + 1 < nch)
                    def _():
                        pltpu.make_async_copy(idx_src(g + 1), ibuf[nb],
                                              isem).wait()
                        issue_gather(nb)
                    pltpu.async_copy(hbuf[b], acc_sh.at[ibuf[b].at[1]],
                                     ssem[b], add=True)
                    pltpu.async_copy(exbuf[b], s_sh.at[ibuf[b].at[1]],
                                     esem[b], add=True)
            return carry
        lax.fori_loop(0, npair, pair, 0)

        wait_scatters((nch - 1) % 2)
        wait_scatters(nch % 2)

        plsc.subcore_barrier()

        chunks = [(i * zrows, zrows) for i in range(rpt // zrows)]
        if rpt % zrows:
            chunks.append(((rpt // zrows) * zrows, rpt % zrows))
        for i, (off, ln) in enumerate(chunks):
            b = i % 2
            if i >= 2:
                poff, pln = chunks[i - 2]
                pltpu.make_async_copy(
                    hbuf[b].at[pl.ds(0, pln)],
                    acc_hbm.at[cid, pl.ds(sid * rpt + poff, pln)],
                    gsem[b]).wait()
            pltpu.sync_copy(acc_sh.at[pl.ds(sid * rpt + off, ln)],
                            hbuf[b].at[pl.ds(0, ln)])
            pltpu.async_copy(hbuf[b].at[pl.ds(0, ln)],
                             acc_hbm.at[cid, pl.ds(sid * rpt + off, ln)],
                             gsem[b])
        for i in (len(chunks) - 2, len(chunks) - 1):
            off, ln = chunks[i]
            pltpu.make_async_copy(
                hbuf[i % 2].at[pl.ds(0, ln)],
                acc_hbm.at[cid, pl.ds(sid * rpt + off, ln)],
                gsem[i % 2]).wait()
        pltpu.sync_copy(s_sh.at[pl.ds(sid * rpt, rpt)], zs.at[pl.ds(0, rpt)])
        for core, s_hbm in ((0, s0_hbm), (1, s1_hbm)):
            @pl.when(cid == core)
            def _():
                pltpu.sync_copy(zs.at[pl.ds(0, rpt)],
                                s_hbm.at[pl.ds(sid * rpt, rpt)])

        @pl.when(sid == NS - 1)
        def _():
            pltpu.sync_copy(acc_sh.at[pl.ds(NS * rpt, tail)],
                            hbuf[0].at[pl.ds(0, tail)])
            pltpu.sync_copy(hbuf[0].at[pl.ds(0, tail)],
                            acc_hbm.at[cid, pl.ds(NS * rpt, tail)])
            pltpu.sync_copy(s_sh.at[pl.ds(NS * rpt, tail)],
                            zs.at[pl.ds(0, tail)])
            for core, s_hbm in ((0, s0_hbm), (1, s1_hbm)):
                @pl.when(cid == core)
                def _():
                    pltpu.sync_copy(zs.at[pl.ds(0, tail)],
                                    s_hbm.at[pl.ds(NS * rpt, tail)])

    return k(h, es, ed, edge_index)


def kernel(x, edge_index, W1, att_src1, att_dst1, b1, W2, att_src2, att_dst2, b2):
    n = x.shape[0]
    hid = W1.shape[1]
    out = W2.shape[1]
    blk = 1000

    h1, es1, ed1 = _tc_layer1(x, W1, att_src1, att_dst1, blk)
    acc1, s1a, s1b = _sc_msgpass(h1, es1.reshape(n), ed1.reshape(n), edge_index)
    h2, es2, ed2 = _tc_layer2(acc1, s1a.reshape(n, 1), s1b.reshape(n, 1),
                              b1.reshape(1, hid), W2, att_src2, att_dst2, blk)
    acc2, s2a, s2b = _sc_msgpass(h2, es2.reshape(n), ed2.reshape(n), edge_index)
    return _tc_final(acc2, s2a.reshape(n, 1), s2b.reshape(n, 1),
                     b2.reshape(1, out), blk)


# speedup vs baseline: 1.0176x; 1.0160x over previous
import functools

import jax
import jax.numpy as jnp
from jax import lax
from jax.experimental import pallas as pl
from jax.experimental.pallas import tpu as pltpu
from jax.experimental.pallas import tpu_sc as plsc

NC = 2
NS = 16
LANES = 16


def _tc_layer1(x, W1, a_src, a_dst, blk):
    n, in_dim = x.shape
    hid = W1.shape[1]
    grid = n // blk

    def body(x_ref, w_ref, asrc_ref, adst_ref, h_ref, es_ref, ed_ref):
        h = jnp.dot(x_ref[...], w_ref[...], preferred_element_type=jnp.float32)
        h_ref[...] = h
        es_ref[...] = jnp.sum(h * asrc_ref[...], axis=1, keepdims=True)
        ed_ref[...] = jnp.sum(h * adst_ref[...], axis=1, keepdims=True)

    return pl.pallas_call(
        body,
        grid=(grid,),
        in_specs=[
            pl.BlockSpec((blk, in_dim), lambda i: (i, 0)),
            pl.BlockSpec((in_dim, hid), lambda i: (0, 0)),
            pl.BlockSpec((1, hid), lambda i: (0, 0)),
            pl.BlockSpec((1, hid), lambda i: (0, 0)),
        ],
        out_specs=[
            pl.BlockSpec((blk, hid), lambda i: (i, 0)),
            pl.BlockSpec((blk, 1), lambda i: (i, 0)),
            pl.BlockSpec((blk, 1), lambda i: (i, 0)),
        ],
        out_shape=[
            jax.ShapeDtypeStruct((n, hid), jnp.float32),
            jax.ShapeDtypeStruct((n, 1), jnp.float32),
            jax.ShapeDtypeStruct((n, 1), jnp.float32),
        ],
    )(x, W1, a_src, a_dst)


def _tc_layer2(acc, s0, s1, b1, W2, a_src, a_dst, blk):
    n = acc.shape[1]
    hid = acc.shape[2]
    out = W2.shape[1]
    grid = n // blk

    def body(acc_ref, s0_ref, s1_ref, b_ref, w_ref, asrc_ref, adst_ref,
             h_ref, es_ref, ed_ref):
        num = acc_ref[0] + acc_ref[1]
        den = s0_ref[...] + s1_ref[...] + 1e-16
        z = num / den + b_ref[...]
        z = jnp.where(z > 0, z, jnp.exp(z) - 1.0)
        h = jnp.dot(z, w_ref[...], preferred_element_type=jnp.float32)
        h_ref[...] = h
        es_ref[...] = jnp.sum(h * asrc_ref[...], axis=1, keepdims=True)
        ed_ref[...] = jnp.sum(h * adst_ref[...], axis=1, keepdims=True)

    return pl.pallas_call(
        body,
        grid=(grid,),
        in_specs=[
            pl.BlockSpec((NC, blk, hid), lambda i: (0, i, 0)),
            pl.BlockSpec((blk, 1), lambda i: (i, 0)),
            pl.BlockSpec((blk, 1), lambda i: (i, 0)),
            pl.BlockSpec((1, hid), lambda i: (0, 0)),
            pl.BlockSpec((hid, out), lambda i: (0, 0)),
            pl.BlockSpec((1, out), lambda i: (0, 0)),
            pl.BlockSpec((1, out), lambda i: (0, 0)),
        ],
        out_specs=[
            pl.BlockSpec((blk, out), lambda i: (i, 0)),
            pl.BlockSpec((blk, 1), lambda i: (i, 0)),
            pl.BlockSpec((blk, 1), lambda i: (i, 0)),
        ],
        out_shape=[
            jax.ShapeDtypeStruct((n, out), jnp.float32),
            jax.ShapeDtypeStruct((n, 1), jnp.float32),
            jax.ShapeDtypeStruct((n, 1), jnp.float32),
        ],
    )(acc, s0, s1, b1, W2, a_src, a_dst)


def _tc_final(acc, s0, s1, b2, blk):
    n = acc.shape[1]
    out = acc.shape[2]
    grid = n // blk

    def body(acc_ref, s0_ref, s1_ref, b_ref, o_ref):
        o = (acc_ref[0] + acc_ref[1]) / (s0_ref[...] + s1_ref[...] + 1e-16)
        o = o + b_ref[...]
        m = jnp.max(o, axis=1, keepdims=True)
        lse = m + jnp.log(jnp.sum(jnp.exp(o - m), axis=1, keepdims=True))
        o_ref[...] = o - lse

    return pl.pallas_call(
        body,
        grid=(grid,),
        in_specs=[
            pl.BlockSpec((NC, blk, out), lambda i: (0, i, 0)),
            pl.BlockSpec((blk, 1), lambda i: (i, 0)),
            pl.BlockSpec((blk, 1), lambda i: (i, 0)),
            pl.BlockSpec((1, out), lambda i: (0, 0)),
        ],
        out_specs=pl.BlockSpec((blk, out), lambda i: (i, 0)),
        out_shape=jax.ShapeDtypeStruct((n, out), jnp.float32),
    )(acc, s0, s1, b2)


def _sc_msgpass(h, es, ed, edge_index):
    n, c = h.shape
    e = edge_index.shape[1]
    nw = NC * NS
    ept = e // nw
    ch = 80
    nch = ept // ch
    npair = (nch + 1) // 2
    rpt = 624
    tail = n - NS * rpt
    cr = c // LANES
    zrows = 64

    mesh = plsc.VectorSubcoreMesh(core_axis_name="c", subcore_axis_name="s",
                                  num_cores=NC, num_subcores=NS)

    @functools.partial(
        pl.kernel,
        out_type=[
            jax.ShapeDtypeStruct((NC, n, c), jnp.float32),
            jax.ShapeDtypeStruct((n,), jnp.float32),
            jax.ShapeDtypeStruct((n,), jnp.float32),
        ],
        mesh=mesh,
        compiler_params=pltpu.CompilerParams(needs_layout_passes=False,
                                             use_tc_tiling_on_sc=False),
        scratch_types=[
            pltpu.VMEM((n,), jnp.float32),
            pltpu.VMEM((n,), jnp.float32),
            [pltpu.VMEM((ch,), jnp.float32)] * 2,
            [pltpu.VMEM((2, ch), jnp.int32)] * 2,
            [pltpu.VMEM((ch, c), jnp.float32)] * 2,
            pltpu.VMEM((rpt + tail,), jnp.float32),
            pltpu.VMEM_SHARED((n, c), jnp.float32),
            pltpu.VMEM_SHARED((n,), jnp.float32),
            [pltpu.SemaphoreType.DMA] * 2,
            [pltpu.SemaphoreType.DMA] * 2,
            [pltpu.SemaphoreType.DMA] * 2,
            pltpu.SemaphoreType.DMA,
        ],
    )
    def k(h_hbm, es_hbm, ed_hbm, ei_hbm, acc_hbm, s0_hbm, s1_hbm,
          es_v, ed_v, exbuf, ibuf, hbuf,
          zs, acc_sh, s_sh, gsem, ssem, esem, isem):
        cid = lax.axis_index("c")
        sid = lax.axis_index("s")
        wid = cid * NS + sid
        ebase = wid * ept
        zero16 = jnp.zeros((LANES,), jnp.float32)

        def idx_src(g):
            return ei_hbm.at[:, pl.ds(ebase + g * ch, ch)]

        def issue_gather(b):
            pltpu.async_copy(h_hbm.at[ibuf[b].at[0]], hbuf[b], gsem[b])

        pltpu.async_copy(es_hbm, es_v, esem[0])
        pltpu.async_copy(ed_hbm, ed_v, esem[1])
        pltpu.async_copy(idx_src(0), ibuf[0], isem)

        def zzs(i, carry):
            zs[pl.ds(i * LANES, LANES)] = zero16
            return carry
        lax.fori_loop(0, (rpt + tail) // LANES, zzs, 0)

        def zzr(i, carry):
            for r in range(cr):
                hbuf[1][i, pl.ds(r * LANES, LANES)] = zero16
            return carry
        lax.fori_loop(0, zrows, zzr, 0)

        nz = rpt // zrows
        rem = rpt % zrows
        for i in range(nz):
            pltpu.async_copy(hbuf[1].at[pl.ds(0, zrows)],
                             acc_sh.at[pl.ds(sid * rpt + i * zrows, zrows)],
                             ssem[0])
        if rem:
            pltpu.async_copy(hbuf[1].at[pl.ds(0, rem)],
                             acc_sh.at[pl.ds(sid * rpt + nz * zrows, rem)],
                             ssem[0])
        pltpu.async_copy(zs.at[pl.ds(0, rpt)],
                         s_sh.at[pl.ds(sid * rpt, rpt)], ssem[1])

        @pl.when(sid == NS - 1)
        def _():
            pltpu.async_copy(hbuf[1].at[pl.ds(0, tail)],
                             acc_sh.at[pl.ds(NS * rpt, tail)], ssem[0])
            pltpu.async_copy(zs.at[pl.ds(0, tail)],
                             s_sh.at[pl.ds(NS * rpt, tail)], ssem[1])

        for i in range(nz):
            pltpu.make_async_copy(hbuf[1].at[pl.ds(0, zrows)],
                                  acc_sh.at[pl.ds(sid * rpt + i * zrows,
                                                  zrows)], ssem[0]).wait()
        if rem:
            pltpu.make_async_copy(hbuf[1].at[pl.ds(0, rem)],
                                  acc_sh.at[pl.ds(sid * rpt + nz * zrows,
                                                  rem)], ssem[0]).wait()
        pltpu.make_async_copy(zs.at[pl.ds(0, rpt)],
                              s_sh.at[pl.ds(sid * rpt, rpt)], ssem[1]).wait()

        @pl.when(sid == NS - 1)
        def _():
            pltpu.make_async_copy(hbuf[1].at[pl.ds(0, tail)],
                                  acc_sh.at[pl.ds(NS * rpt, tail)],
                                  ssem[0]).wait()
            pltpu.make_async_copy(zs.at[pl.ds(0, tail)],
                                  s_sh.at[pl.ds(NS * rpt, tail)],
                                  ssem[1]).wait()

        pltpu.make_async_copy(idx_src(0), ibuf[0], isem).wait()
        issue_gather(0)
        pltpu.make_async_copy(es_hbm, es_v, esem[0]).wait()
        pltpu.make_async_copy(ed_hbm, ed_v, esem[1]).wait()

        plsc.subcore_barrier()

        def compute_ex(b):
            @plsc.parallel_loop(0, ch // LANES, unroll=2)
            def lanes(j):
                sidx = ibuf[b][0, pl.ds(j * LANES, LANES)]
                didx = ibuf[b][1, pl.ds(j * LANES, LANES)]
                ea = plsc.load_gather(es_v, [sidx])
                eb = plsc.load_gather(ed_v, [didx])
                ee = ea + eb
                ee = jnp.where(ee >= 0, ee, 0.2 * ee)
                exbuf[b][pl.ds(j * LANES, LANES)] = jnp.exp(ee)

        def scale(b):
            @plsc.parallel_loop(0, ch // LANES, unroll=2)
            def sbody(j):
                exvec = exbuf[b][pl.ds(j * LANES, LANES)]
                for i in range(LANES):
                    al = exvec[i]
                    row = j * LANES + i
                    for r in range(cr):
                        sl = pl.ds(r * LANES, LANES)
                        hbuf[b][row, sl] = hbuf[b][row, sl] * al

        def wait_gather(b):
            pltpu.make_async_copy(h_hbm.at[ibuf[b].at[0]], hbuf[b],
                                  gsem[b]).wait()

        def wait_scatters(b):
            pltpu.make_async_copy(hbuf[b], acc_sh.at[ibuf[b].at[1]],
                                  ssem[b]).wait()
            pltpu.make_async_copy(exbuf[b], s_sh.at[ibuf[b].at[1]],
                                  esem[b]).wait()

        def pair(g2, carry):
            for b in (0, 1):
                g = 2 * g2 + b
                nb = 1 - b

                @pl.when(g < nch)
                def _():
                    compute_ex(b)

                    @pl.when((g >= 1) & (g + 1 < nch))
                    def _():
                        wait_scatters(nb)

                    @pl.when(g + 1 < nch)
                    def _():
                        pltpu.async_copy(idx_src(g + 1), ibuf[nb], isem)
                    wait_gather(b)
                    scale(b)

                    @pl.when(g + 1 < nch)
                    def _():
                        pltpu.make_async_copy(idx_src(g + 1), ibuf[nb],
                                              isem).wait()
                        issue_gather(nb)
                    pltpu.async_copy(hbuf[b], acc_sh.at[ibuf[b].at[1]],
                                     ssem[b], add=True)
                    pltpu.async_copy(exbuf[b], s_sh.at[ibuf[b].at[1]],
                                     esem[b], add=True)
            return carry
        lax.fori_loop(0, npair, pair, 0)

        wait_scatters((nch - 1) % 2)
        wait_scatters(nch % 2)

        plsc.subcore_barrier()

        chunks = [(i * zrows, zrows) for i in range(rpt // zrows)]
        if rpt % zrows:
            chunks.append(((rpt // zrows) * zrows, rpt % zrows))
        for i, (off, ln) in enumerate(chunks):
            b = i % 2
            if i >= 2:
                poff, pln = chunks[i - 2]
                pltpu.make_async_copy(
                    hbuf[b].at[pl.ds(0, pln)],
                    acc_hbm.at[cid, pl.ds(sid * rpt + poff, pln)],
                    gsem[b]).wait()
            pltpu.sync_copy(acc_sh.at[pl.ds(sid * rpt + off, ln)],
                            hbuf[b].at[pl.ds(0, ln)])
            pltpu.async_copy(hbuf[b].at[pl.ds(0, ln)],
                             acc_hbm.at[cid, pl.ds(sid * rpt + off, ln)],
                             gsem[b])
        for i in (len(chunks) - 2, len(chunks) - 1):
            off, ln = chunks[i]
            pltpu.make_async_copy(
                hbuf[i % 2].at[pl.ds(0, ln)],
                acc_hbm.at[cid, pl.ds(sid * rpt + off, ln)],
                gsem[i % 2]).wait()
        pltpu.sync_copy(s_sh.at[pl.ds(sid * rpt, rpt)], zs.at[pl.ds(0, rpt)])
        for core, s_hbm in ((0, s0_hbm), (1, s1_hbm)):
            @pl.when(cid == core)
            def _():
                pltpu.sync_copy(zs.at[pl.ds(0, rpt)],
                                s_hbm.at[pl.ds(sid * rpt, rpt)])

        @pl.when(sid == NS - 1)
        def _():
            pltpu.sync_copy(acc_sh.at[pl.ds(NS * rpt, tail)],
                            hbuf[0].at[pl.ds(0, tail)])
            pltpu.sync_copy(hbuf[0].at[pl.ds(0, tail)],
                            acc_hbm.at[cid, pl.ds(NS * rpt, tail)])
            pltpu.sync_copy(s_sh.at[pl.ds(NS * rpt, tail)],
                            zs.at[pl.ds(0, tail)])
            for core, s_hbm in ((0, s0_hbm), (1, s1_hbm)):
                @pl.when(cid == core)
                def _():
                    pltpu.sync_copy(zs.at[pl.ds(0, tail)],
                                    s_hbm.at[pl.ds(NS * rpt, tail)])

    return k(h, es, ed, edge_index)


def kernel(x, edge_index, W1, att_src1, att_dst1, b1, W2, att_src2, att_dst2, b2):
    n = x.shape[0]
    hid = W1.shape[1]
    out = W2.shape[1]
    blk = 2000

    h1, es1, ed1 = _tc_layer1(x, W1, att_src1, att_dst1, blk)
    acc1, s1a, s1b = _sc_msgpass(h1, es1.reshape(n), ed1.reshape(n), edge_index)
    h2, es2, ed2 = _tc_layer2(acc1, s1a.reshape(n, 1), s1b.reshape(n, 1),
                              b1.reshape(1, hid), W2, att_src2, att_dst2, blk)
    acc2, s2a, s2b = _sc_msgpass(h2, es2.reshape(n), ed2.reshape(n), edge_index)
    return _tc_final(acc2, s2a.reshape(n, 1), s2b.reshape(n, 1),
                     b2.reshape(1, out), blk)
